# baseline (speedup 1.0000x reference)
import functools

import jax
import jax.numpy as jnp
from jax import lax
from jax.experimental import pallas as pl
from jax.experimental.pallas import tpu as pltpu
from jax.experimental.pallas import tpu_sc as plsc

VOCAB = 100000
WORD_DIM = 252
TEMPL_DIM = 256
OUT_DIM = 512
B, L = 1024, 200
N = B * L
NC, NS = 2, 16
NW = NC * NS
PER_W = N // NW
C = 128
NCHUNK = PER_W // C
BM = 1024


def _make_gather_kernel():
    mesh = plsc.VectorSubcoreMesh(core_axis_name="c", subcore_axis_name="s")

    @functools.partial(
        pl.kernel,
        mesh=mesh,
        out_type=(
            jax.ShapeDtypeStruct((N, TEMPL_DIM), jnp.float32),
            jax.ShapeDtypeStruct((N, TEMPL_DIM), jnp.float32),
        ),
        scratch_types=[
            pltpu.VMEM((C,), jnp.int32),
            pltpu.VMEM((C,), jnp.int32),
            pltpu.VMEM((C, TEMPL_DIM), jnp.float32),
            pltpu.VMEM((C, TEMPL_DIM), jnp.float32),
            pltpu.SemaphoreType.DMA,
        ],
        compiler_params=pltpu.CompilerParams(use_tc_tiling_on_sc=False),
    )
    def gather2(cat_hbm, templ_hbm, ww_hbm, wt_hbm, gw_hbm, gt_hbm,
                idx_w, idx_t, buf_w, buf_t, sem):
        wid = lax.axis_index("s") * NC + lax.axis_index("c")
        base0 = wid * PER_W

        def body(i, carry):
            base = base0 + i * C
            pltpu.sync_copy(cat_hbm.at[pl.ds(base, C)], idx_w)
            pltpu.sync_copy(templ_hbm.at[pl.ds(base, C)], idx_t)
            cw = pltpu.async_copy(ww_hbm.at[idx_w], buf_w, sem)
            ct = pltpu.async_copy(wt_hbm.at[idx_t], buf_t, sem)
            cw.wait()
            ct.wait()
            pltpu.sync_copy(buf_w, gw_hbm.at[pl.ds(base, C)])
            pltpu.sync_copy(buf_t, gt_hbm.at[pl.ds(base, C)])
            return carry

        lax.fori_loop(0, NCHUNK, body, 0)

    return gather2


_gather2 = _make_gather_kernel()


def _concat_body(a_ref, b_ref, c_ref, o_ref):
    o_ref[...] = jnp.concatenate(
        [a_ref[:, :WORD_DIM], b_ref[...], c_ref[...]], axis=-1)


_concat_call = pl.pallas_call(
    _concat_body,
    grid=(N // BM,),
    in_specs=[
        pl.BlockSpec((BM, TEMPL_DIM), lambda i: (i, 0)),
        pl.BlockSpec((BM, TEMPL_DIM), lambda i: (i, 0)),
        pl.BlockSpec((BM, 4), lambda i: (i, 0)),
    ],
    out_specs=pl.BlockSpec((BM, OUT_DIM), lambda i: (i, 0)),
    out_shape=jax.ShapeDtypeStruct((N, OUT_DIM), jnp.float32),
)


def kernel(cat_ids, box, template, W_word, W_templ):
    cat_flat = cat_ids.reshape(N).astype(jnp.int32)
    templ_flat = template.reshape(N).astype(jnp.int32)
    box_flat = box.reshape(N, 4)
    ww_pad = jnp.pad(W_word, ((0, 0), (0, TEMPL_DIM - WORD_DIM)))
    gw, gt = _gather2(cat_flat, templ_flat, ww_pad, W_templ)
    out = _concat_call(gw, gt, box_flat)
    return out.reshape(B, L, OUT_DIM)


# speedup vs baseline: 1.0326x; 1.0326x over previous
import functools

import jax
import jax.numpy as jnp
from jax import lax
from jax.experimental import pallas as pl
from jax.experimental.pallas import tpu as pltpu
from jax.experimental.pallas import tpu_sc as plsc

VOCAB = 100000
WORD_DIM = 252
TEMPL_DIM = 256
OUT_DIM = 512
B, L = 1024, 200
N = B * L
NC, NS = 2, 16
NW = NC * NS
PER_W = N // NW
C = 128
NCHUNK = PER_W // C
SEAM0 = WORD_DIM - 4
SEAM1 = OUT_DIM - 8


def _make_sc_kernel():
    mesh = plsc.VectorSubcoreMesh(core_axis_name="c", subcore_axis_name="s")

    @functools.partial(
        pl.kernel,
        mesh=mesh,
        out_type=jax.ShapeDtypeStruct((N, OUT_DIM), jnp.float32),
        scratch_types=[
            pltpu.VMEM((C,), jnp.int32),
            pltpu.VMEM((C,), jnp.int32),
            pltpu.VMEM((C, TEMPL_DIM), jnp.float32),
            pltpu.VMEM((C, TEMPL_DIM), jnp.float32),
            pltpu.VMEM((4 * C,), jnp.float32),
            pltpu.VMEM((C, 8), jnp.float32),
            pltpu.VMEM((C, 8), jnp.float32),
            pltpu.SemaphoreType.DMA,
        ],
        compiler_params=pltpu.CompilerParams(
            use_tc_tiling_on_sc=False, needs_layout_passes=False),
    )
    def emb_concat(cat_hbm, templ_hbm, box_hbm, ww_hbm, wt_hbm, out_hbm,
                   idx_w, idx_t, buf_w, buf_t, buf_b, mid0, mid1, sem):
        wid = lax.axis_index("s") * NC + lax.axis_index("c")
        base0 = wid * PER_W

        def body(i, carry):
            lane = lax.iota(jnp.int32, 16)
            row4 = lax.shift_right_logical(lane, 2)
            col4 = lax.bitwise_and(lane, 3)
            base = base0 + i * C
            pltpu.sync_copy(cat_hbm.at[pl.ds(base, C)], idx_w)
            pltpu.sync_copy(templ_hbm.at[pl.ds(base, C)], idx_t)
            cw = pltpu.async_copy(ww_hbm.at[idx_w], buf_w, sem)
            ct = pltpu.async_copy(wt_hbm.at[idx_t], buf_t, sem)
            cb = pltpu.async_copy(
                box_hbm.at[pl.ds(base * 4, 4 * C)], buf_b, sem)
            cw.wait()
            ct.wait()
            cb.wait()
            for g in range(C // 4):
                rows = row4 + (4 * g)
                wt4 = plsc.load_gather(buf_w, [rows, col4 + SEAM0])
                plsc.store_scatter(mid0, [rows, col4], wt4)
                th4 = plsc.load_gather(buf_t, [rows, col4 + (TEMPL_DIM - 4)])
                plsc.store_scatter(mid0, [rows, col4 + 4], th4)
                tt4 = plsc.load_gather(buf_t, [rows, col4 + (TEMPL_DIM - 8)])
                plsc.store_scatter(mid1, [rows, col4], tt4)
                bx = buf_b[pl.ds(16 * g, 16)]
                plsc.store_scatter(mid1, [rows, col4 + 4], bx)
            pltpu.sync_copy(
                buf_w.at[:, pl.ds(0, SEAM0)],
                out_hbm.at[pl.ds(base, C), pl.ds(0, SEAM0)])
            pltpu.sync_copy(
                mid0, out_hbm.at[pl.ds(base, C), pl.ds(SEAM0, 8)])
            pltpu.sync_copy(
                buf_t.at[:, pl.ds(0, SEAM0)],
                out_hbm.at[pl.ds(base, C), pl.ds(TEMPL_DIM, SEAM0)])
            pltpu.sync_copy(
                mid1, out_hbm.at[pl.ds(base, C), pl.ds(SEAM1, 8)])
            return carry

        lax.fori_loop(0, NCHUNK, body, 0)

    return emb_concat


_emb_concat = _make_sc_kernel()


def kernel(cat_ids, box, template, W_word, W_templ):
    cat_flat = cat_ids.reshape(N).astype(jnp.int32)
    templ_flat = template.reshape(N).astype(jnp.int32)
    box_flat = box.reshape(N * 4)
    ww_pad = jnp.pad(W_word, ((0, 0), (0, TEMPL_DIM - WORD_DIM)))
    wt_rot = jnp.concatenate([W_templ[:, 4:], W_templ[:, :4]], axis=1)
    out = _emb_concat(cat_flat, templ_flat, box_flat, ww_pad, wt_rot)
    return out.reshape(B, L, OUT_DIM)


# speedup vs baseline: 1.8224x; 1.7648x over previous
import functools

import jax
import jax.numpy as jnp
from jax import lax
from jax.experimental import pallas as pl
from jax.experimental.pallas import tpu as pltpu
from jax.experimental.pallas import tpu_sc as plsc

VOCAB = 100000
WORD_DIM = 252
TEMPL_DIM = 256
OUT_DIM = 512
B, L = 1024, 200
N = B * L
NC, NS = 2, 16
NW = NC * NS
PER_W = N // NW
C = 128
NCHUNK = PER_W // C


def _make_sc_kernel():
    mesh = plsc.VectorSubcoreMesh(core_axis_name="c", subcore_axis_name="s")

    @functools.partial(
        pl.kernel,
        mesh=mesh,
        out_type=jax.ShapeDtypeStruct((N, OUT_DIM), jnp.float32),
        scratch_types=[
            pltpu.VMEM((C,), jnp.int32),
            pltpu.VMEM((C,), jnp.int32),
            pltpu.VMEM((C, OUT_DIM), jnp.float32),
            pltpu.VMEM((4 * C,), jnp.float32),
            pltpu.SemaphoreType.DMA,
        ],
        compiler_params=pltpu.CompilerParams(needs_layout_passes=False),
    )
    def emb_concat(cat_hbm, templ_hbm, box_hbm, ww_hbm, wt_hbm, out_hbm,
                   idx_w, idx_t, rows, buf_b, sem):
        wid = lax.axis_index("s") * NC + lax.axis_index("c")
        base0 = wid * PER_W

        def body(i, carry):
            base = base0 + i * C
            pltpu.sync_copy(cat_hbm.at[pl.ds(base, C)], idx_w)
            pltpu.sync_copy(templ_hbm.at[pl.ds(base, C)], idx_t)
            cw = pltpu.async_copy(
                ww_hbm.at[idx_w], rows.at[:, pl.ds(0, TEMPL_DIM)], sem)
            ct = pltpu.async_copy(
                wt_hbm.at[idx_t], rows.at[:, pl.ds(TEMPL_DIM, TEMPL_DIM)],
                sem)
            cb = pltpu.async_copy(
                box_hbm.at[pl.ds(base * 4, 4 * C)], buf_b, sem)
            cw.wait()
            ct.wait()
            cb.wait()
            lane = lax.iota(jnp.int32, 16)
            row4 = lax.shift_right_logical(lane, 2)
            col4 = lax.bitwise_and(lane, 3)
            for g in range(C // 4):
                rg = row4 + (4 * g)
                th = plsc.load_gather(rows, [rg, col4 + (OUT_DIM - 4)])
                plsc.store_scatter(rows, [rg, col4 + WORD_DIM], th)
                bx = buf_b[pl.ds(16 * g, 16)]
                plsc.store_scatter(rows, [rg, col4 + (OUT_DIM - 4)], bx)
            pltpu.sync_copy(rows, out_hbm.at[pl.ds(base, C)])
            return carry

        lax.fori_loop(0, NCHUNK, body, 0)

    return emb_concat


_emb_concat = _make_sc_kernel()


def kernel(cat_ids, box, template, W_word, W_templ):
    cat_flat = cat_ids.reshape(N).astype(jnp.int32)
    templ_flat = template.reshape(N).astype(jnp.int32)
    box_flat = box.reshape(N * 4)
    ww_pad = jnp.pad(W_word, ((0, 0), (0, TEMPL_DIM - WORD_DIM)))
    wt_rot = jnp.concatenate([W_templ[:, 4:], W_templ[:, :4]], axis=1)
    out = _emb_concat(cat_flat, templ_flat, box_flat, ww_pad, wt_rot)
    return out.reshape(B, L, OUT_DIM)


# speedup vs baseline: 2.0205x; 1.1087x over previous
import functools

import jax
import jax.numpy as jnp
from jax import lax
from jax.experimental import pallas as pl
from jax.experimental.pallas import tpu as pltpu
from jax.experimental.pallas import tpu_sc as plsc

VOCAB = 100000
WORD_DIM = 252
TEMPL_DIM = 256
OUT_DIM = 512
B, L = 1024, 200
N = B * L
NC, NS = 2, 16
NW = NC * NS
PER_W = N // NW
C = 80
NCHUNK = PER_W // C
NPAIR = NCHUNK // 2


def _make_sc_kernel():
    mesh = plsc.VectorSubcoreMesh(core_axis_name="c", subcore_axis_name="s")

    @functools.partial(
        pl.kernel,
        mesh=mesh,
        out_type=jax.ShapeDtypeStruct((N, OUT_DIM), jnp.float32),
        scratch_types=[
            pltpu.VMEM((C,), jnp.int32),
            pltpu.VMEM((C,), jnp.int32),
            pltpu.VMEM((C,), jnp.int32),
            pltpu.VMEM((C,), jnp.int32),
            pltpu.VMEM((C, OUT_DIM), jnp.float32),
            pltpu.VMEM((C, OUT_DIM), jnp.float32),
            pltpu.VMEM((4 * C,), jnp.float32),
            pltpu.VMEM((4 * C,), jnp.float32),
            pltpu.SemaphoreType.DMA,
            pltpu.SemaphoreType.DMA,
            pltpu.SemaphoreType.DMA,
            pltpu.SemaphoreType.DMA,
        ],
        compiler_params=pltpu.CompilerParams(needs_layout_passes=False),
    )
    def emb_concat(cat_hbm, templ_hbm, box_hbm, ww_hbm, wt_hbm, out_hbm,
                   idx_w0, idx_t0, idx_w1, idx_t1, rows0, rows1,
                   bb0, bb1, sg0, sg1, si0, si1):
        wid = lax.axis_index("s") * NC + lax.axis_index("c")
        base0 = wid * PER_W
        idx_sets = ((idx_w0, idx_t0, rows0, bb0, sg0, si0),
                    (idx_w1, idx_t1, rows1, bb1, sg1, si1))

        def stage_idx(c, st):
            iw, it, _, _, _, si = st
            base = base0 + c * C
            pltpu.async_copy(cat_hbm.at[pl.ds(base, C)], iw, si)
            pltpu.async_copy(templ_hbm.at[pl.ds(base, C)], it, si)

        def wait_idx(c, st):
            iw, it, _, _, _, si = st
            base = base0 + c * C
            pltpu.make_async_copy(cat_hbm.at[pl.ds(base, C)], iw, si).wait()
            pltpu.make_async_copy(
                templ_hbm.at[pl.ds(base, C)], it, si).wait()

        def start_gathers(c, st):
            iw, it, rows, bb, sg, _ = st
            base = base0 + c * C
            pltpu.async_copy(
                ww_hbm.at[iw], rows.at[:, pl.ds(0, TEMPL_DIM)], sg)
            pltpu.async_copy(
                wt_hbm.at[it], rows.at[:, pl.ds(TEMPL_DIM, TEMPL_DIM)], sg)
            pltpu.async_copy(box_hbm.at[pl.ds(base * 4, 4 * C)], bb, sg)

        def wait_gathers(c, st):
            _, _, rows, bb, sg, _ = st
            pltpu.make_async_copy(
                out_hbm.at[pl.ds(0, C), pl.ds(0, TEMPL_DIM)],
                rows.at[:, pl.ds(0, TEMPL_DIM)], sg).wait()
            pltpu.make_async_copy(
                out_hbm.at[pl.ds(0, C), pl.ds(0, TEMPL_DIM)],
                rows.at[:, pl.ds(TEMPL_DIM, TEMPL_DIM)], sg).wait()
            pltpu.make_async_copy(
                box_hbm.at[pl.ds(0, 4 * C)], bb, sg).wait()

        def fixup_and_write(c, st):
            _, _, rows, bb, _, _ = st
            base = base0 + c * C
            lane = lax.iota(jnp.int32, 16)
            row4 = lax.shift_right_logical(lane, 2)
            col4 = lax.bitwise_and(lane, 3)
            for g in range(C // 4):
                rg = row4 + (4 * g)
                th = plsc.load_gather(rows, [rg, col4 + (OUT_DIM - 4)])
                plsc.store_scatter(rows, [rg, col4 + WORD_DIM], th)
                bx = bb[pl.ds(16 * g, 16)]
                plsc.store_scatter(rows, [rg, col4 + (OUT_DIM - 4)], bx)
            pltpu.sync_copy(rows, out_hbm.at[pl.ds(base, C)])

        stage_idx(0, idx_sets[0])
        wait_idx(0, idx_sets[0])
        start_gathers(0, idx_sets[0])
        stage_idx(1, idx_sets[1])

        def body(p, carry):
            c0 = 2 * p
            c1 = c0 + 1
            last = p >= NPAIR - 1
            wait_idx(c1, idx_sets[1])
            start_gathers(c1, idx_sets[1])
            wait_gathers(c0, idx_sets[0])

            @pl.when(jnp.logical_not(last))
            def _():
                stage_idx(c1 + 1, idx_sets[0])

            fixup_and_write(c0, idx_sets[0])

            @pl.when(jnp.logical_not(last))
            def _():
                wait_idx(c1 + 1, idx_sets[0])
                start_gathers(c1 + 1, idx_sets[0])

            wait_gathers(c1, idx_sets[1])

            @pl.when(jnp.logical_not(last))
            def _():
                stage_idx(c1 + 2, idx_sets[1])

            fixup_and_write(c1, idx_sets[1])
            return carry

        lax.fori_loop(0, NPAIR, body, 0)

    return emb_concat


_emb_concat = _make_sc_kernel()


def kernel(cat_ids, box, template, W_word, W_templ):
    cat_flat = cat_ids.reshape(N).astype(jnp.int32)
    templ_flat = template.reshape(N).astype(jnp.int32)
    box_flat = box.reshape(N * 4)
    ww_pad = jnp.pad(W_word, ((0, 0), (0, TEMPL_DIM - WORD_DIM)))
    wt_rot = jnp.concatenate([W_templ[:, 4:], W_templ[:, :4]], axis=1)
    out = _emb_concat(cat_flat, templ_flat, box_flat, ww_pad, wt_rot)
    return out.reshape(B, L, OUT_DIM)
